# baseline (speedup 1.0000x reference)
import functools

import jax
import jax.numpy as jnp
from jax import lax
from jax.experimental import pallas as pl
from jax.experimental.pallas import tpu as pltpu
from jax.experimental.pallas import tpu_sc as plsc

N = 10000
E = 320000
IN_CH = 128
HIDDEN = 128
HID2 = 64

NC = 2
NS = 16
NP = 10112
RPS = NP // NS
CHUNK = 128
CPS = 80
EPS = CHUNK * CPS
EP = EPS * NC * NS
HSTEP = EPS // 16
HCPS = CPS // 2


def _deg_body(dst_hbm, deg_out, idx_v, hist_v):
    c = lax.axis_index("c")
    s = lax.axis_index("s")
    pltpu.sync_copy(dst_hbm.at[c, s], idx_v)

    def zero_body(i, carry):
        hist_v[pl.ds(i * 16, 16)] = jnp.zeros((16,), jnp.float32)
        return carry

    lax.fori_loop(0, NP // 16, zero_body, 0)

    ones = jnp.ones((16,), jnp.float32)

    def hist_body(i, carry):
        idx16 = idx_v[pl.ds(i * 16, 16)]
        plsc.addupdate_scatter(hist_v, [idx16], ones)
        return carry

    lax.fori_loop(0, HSTEP, hist_body, 0)
    pltpu.sync_copy(hist_v, deg_out.at[c, s])


_SC_PARAMS = pltpu.CompilerParams(needs_layout_passes=False,
                                  use_tc_tiling_on_sc=False)


def _make_deg_kernel():
    return pl.kernel(
        _deg_body,
        out_type=jax.ShapeDtypeStruct((NC, NS, NP), jnp.float32),
        mesh=plsc.VectorSubcoreMesh(core_axis_name="c", subcore_axis_name="s"),
        scratch_types=[
            pltpu.VMEM((EPS,), jnp.int32),
            pltpu.VMEM((NP,), jnp.float32),
        ],
        compiler_params=_SC_PARAMS,
    )


def _scatter_body(g_hbm, src_hbm, dst_hbm, zeros_hbm, s_out,
                  idx_s, idx_d, rows0, acc, sem0):
    c = lax.axis_index("c")
    s = lax.axis_index("s")
    pltpu.sync_copy(zeros_hbm.at[pl.ds(s * RPS, RPS)],
                    acc.at[pl.ds(s * RPS, RPS)])
    plsc.subcore_barrier()

    pltpu.sync_copy(src_hbm.at[c, s], idx_s)
    pltpu.sync_copy(dst_hbm.at[c, s], idx_d)

    def chunk_body(j, carry):
        pltpu.async_copy(g_hbm.at[idx_s.at[j]], rows0, sem0).wait()
        pltpu.sync_copy(rows0, acc.at[idx_d.at[j]], add=True)
        return carry

    lax.fori_loop(0, CPS, chunk_body, 0)
    plsc.subcore_barrier()
    pltpu.sync_copy(acc.at[pl.ds(s * RPS, RPS)],
                    s_out.at[c, pl.ds(s * RPS, RPS)])


def _make_scatter_kernel(d):
    return pl.kernel(
        functools.partial(_scatter_body),
        out_type=jax.ShapeDtypeStruct((NC, NP, d), jnp.float32),
        mesh=plsc.VectorSubcoreMesh(core_axis_name="c", subcore_axis_name="s"),
        scratch_types=[
            pltpu.VMEM((CPS, CHUNK), jnp.int32),
            pltpu.VMEM((CPS, CHUNK), jnp.int32),
            pltpu.VMEM((CHUNK, d), jnp.float32),
            pltpu.VMEM_SHARED((NP, d), jnp.float32),
            pltpu.SemaphoreType.DMA,
        ],
        compiler_params=_SC_PARAMS,
    )


def _tc_pre_body(xp_ref, w1_ref, degt_ref, g1_ref):
    deg = jnp.sum(degt_ref[...], axis=1, keepdims=True) + 1.0
    dinv = lax.rsqrt(deg)
    h = lax.dot_general(xp_ref[...], w1_ref[...], (((1,), (1,)), ((), ())),
                        preferred_element_type=jnp.float32)
    g1_ref[...] = h * dinv


def _tc_mid_body(s1_ref, g1_ref, degt_ref, w2_ref, b1_ref, g2_ref):
    deg = jnp.sum(degt_ref[...], axis=1, keepdims=True) + 1.0
    dinv = lax.rsqrt(deg)
    agg = dinv * (s1_ref[0] + s1_ref[1] + g1_ref[...]) + b1_ref[...]
    h1 = jnp.maximum(agg, 0.0)
    h2 = lax.dot_general(h1, w2_ref[...], (((1,), (1,)), ((), ())),
                         preferred_element_type=jnp.float32)
    g2_ref[...] = h2 * dinv


def _tc_post_body(s2_ref, g2_ref, degt_ref, b2_ref, wh_ref, bh_ref, out_ref):
    deg = jnp.sum(degt_ref[...], axis=1, keepdims=True) + 1.0
    dinv = lax.rsqrt(deg)
    agg = dinv * (s2_ref[0] + s2_ref[1] + g2_ref[...]) + b2_ref[...]
    h2 = jnp.maximum(agg, 0.0)
    h2b = h2.astype(jnp.bfloat16).astype(jnp.float32)
    whb = wh_ref[...].astype(jnp.bfloat16).astype(jnp.float32)
    out_ref[...] = jnp.sum(h2b * whb, axis=1, keepdims=True) + bh_ref[...]


def kernel(x, edge_index, W1, b1, W2, b2, Wh, bh):
    f32 = jnp.float32
    src = edge_index[0]
    dst = edge_index[1]
    pad = jnp.full((EP - E,), N, jnp.int32)
    src_p = jnp.concatenate([src, pad]).reshape(NC, NS, CPS, CHUNK)
    dst_p = jnp.concatenate([dst, pad])
    dst_h = dst_p.reshape(NC, NS, EPS)
    dst_c = dst_p.reshape(NC, NS, CPS, CHUNK)
    xp = jnp.pad(x, ((0, NP - N), (0, 0)))

    deg_parts = _make_deg_kernel()(dst_h)
    degt = deg_parts.reshape(NC * NS, NP).T

    g1 = pl.pallas_call(
        _tc_pre_body,
        out_shape=jax.ShapeDtypeStruct((NP, HIDDEN), f32),
    )(xp, W1, degt)

    s1 = _make_scatter_kernel(HIDDEN)(
        g1, src_p, dst_c, jnp.zeros((NP, HIDDEN), f32))

    g2 = pl.pallas_call(
        _tc_mid_body,
        out_shape=jax.ShapeDtypeStruct((NP, HID2), f32),
    )(s1, g1, degt, W2, b1.reshape(1, HIDDEN))

    s2 = _make_scatter_kernel(HID2)(
        g2, src_p, dst_c, jnp.zeros((NP, HID2), f32))

    out = pl.pallas_call(
        _tc_post_body,
        out_shape=jax.ShapeDtypeStruct((NP, 1), f32),
    )(s2, g2, degt, b2.reshape(1, HID2), Wh, bh.reshape(1, 1))

    return out[:N, 0]


# speedup vs baseline: 2.5804x; 2.5804x over previous
import functools

import jax
import jax.numpy as jnp
from jax import lax
from jax.experimental import pallas as pl
from jax.experimental.pallas import tpu as pltpu
from jax.experimental.pallas import tpu_sc as plsc

N = 10000
E = 320000
IN_CH = 128
HIDDEN = 128
HID2 = 64

NC = 2
NS = 16
NP = 10112
RPS = NP // NS
CHUNK = 128
CPS = 79
EPS = CHUNK * CPS
EP = EPS * NC * NS
HSTEP = EPS // 16
HCPS = CPS // 2


def _deg_body(dst_hbm, deg_out, idx_v, hist_v):
    c = lax.axis_index("c")
    s = lax.axis_index("s")
    pltpu.sync_copy(dst_hbm.at[c, s], idx_v)

    def zero_body(i, carry):
        hist_v[pl.ds(i * 16, 16)] = jnp.zeros((16,), jnp.float32)
        return carry

    lax.fori_loop(0, NP // 16, zero_body, 0)

    ones = jnp.ones((16,), jnp.float32)

    def hist_body(i, carry):
        idx16 = idx_v[pl.ds(i * 16, 16)]
        plsc.addupdate_scatter(hist_v, [idx16], ones)
        return carry

    lax.fori_loop(0, HSTEP, hist_body, 0)
    pltpu.sync_copy(hist_v, deg_out.at[c, s])


_SC_PARAMS = pltpu.CompilerParams(needs_layout_passes=False,
                                  use_tc_tiling_on_sc=False)


def _make_deg_kernel():
    return pl.kernel(
        _deg_body,
        out_type=jax.ShapeDtypeStruct((NC, NS, NP), jnp.float32),
        mesh=plsc.VectorSubcoreMesh(core_axis_name="c", subcore_axis_name="s"),
        scratch_types=[
            pltpu.VMEM((EPS,), jnp.int32),
            pltpu.VMEM((NP,), jnp.float32),
        ],
        compiler_params=_SC_PARAMS,
    )


def _scatter_body(g_hbm, src_hbm, dst_hbm, zeros_hbm, s_out,
                  idx_s, idx_d, rows0, acc, sem0):
    c = lax.axis_index("c")
    s = lax.axis_index("s")
    pltpu.sync_copy(zeros_hbm.at[pl.ds(s * RPS, RPS)],
                    acc.at[pl.ds(s * RPS, RPS)])
    plsc.subcore_barrier()

    pltpu.sync_copy(src_hbm.at[c, s], idx_s)
    pltpu.sync_copy(dst_hbm.at[c, s], idx_d)

    def chunk_body(j, carry):
        pltpu.async_copy(g_hbm.at[idx_s.at[j]], rows0, sem0).wait()
        pltpu.sync_copy(rows0, acc.at[idx_d.at[j]], add=True)
        return carry

    lax.fori_loop(0, CPS, chunk_body, 0)
    plsc.subcore_barrier()
    pltpu.sync_copy(acc.at[pl.ds(s * RPS, RPS)],
                    s_out.at[c, pl.ds(s * RPS, RPS)])


def _make_scatter_kernel(d):
    return pl.kernel(
        functools.partial(_scatter_body),
        out_type=jax.ShapeDtypeStruct((NC, NP, d), jnp.float32),
        mesh=plsc.VectorSubcoreMesh(core_axis_name="c", subcore_axis_name="s"),
        scratch_types=[
            pltpu.VMEM((CPS, CHUNK), jnp.int32),
            pltpu.VMEM((CPS, CHUNK), jnp.int32),
            pltpu.VMEM((CHUNK, d), jnp.float32),
            pltpu.VMEM_SHARED((NP, d), jnp.float32),
            pltpu.SemaphoreType.DMA,
        ],
        compiler_params=_SC_PARAMS,
    )


def _tc_pre_body(xp_ref, w1_ref, degt_ref, g1_ref):
    deg = jnp.sum(degt_ref[...], axis=1, keepdims=True) + 1.0
    dinv = lax.rsqrt(deg)
    h = lax.dot_general(xp_ref[...], w1_ref[...], (((1,), (1,)), ((), ())),
                        preferred_element_type=jnp.float32)
    g1_ref[...] = h * dinv


def _tc_mid_body(s1_ref, g1_ref, degt_ref, w2_ref, b1_ref, g2_ref):
    deg = jnp.sum(degt_ref[...], axis=1, keepdims=True) + 1.0
    dinv = lax.rsqrt(deg)
    agg = dinv * (s1_ref[0] + s1_ref[1] + g1_ref[...]) + b1_ref[...]
    h1 = jnp.maximum(agg, 0.0)
    h2 = lax.dot_general(h1, w2_ref[...], (((1,), (1,)), ((), ())),
                         preferred_element_type=jnp.float32)
    g2_ref[...] = h2 * dinv


def _tc_post_body(s2_ref, g2_ref, degt_ref, b2_ref, wh_ref, bh_ref, out_ref):
    deg = jnp.sum(degt_ref[...], axis=1, keepdims=True) + 1.0
    dinv = lax.rsqrt(deg)
    agg = dinv * (s2_ref[0] + s2_ref[1] + g2_ref[...]) + b2_ref[...]
    h2 = jnp.maximum(agg, 0.0)
    h2b = h2.astype(jnp.bfloat16).astype(jnp.float32)
    whb = wh_ref[...].astype(jnp.bfloat16).astype(jnp.float32)
    out_ref[...] = jnp.sum(h2b * whb, axis=1, keepdims=True) + bh_ref[...]


def kernel(x, edge_index, W1, b1, W2, b2, Wh, bh):
    f32 = jnp.float32
    src = edge_index[0]
    dst = edge_index[1]
    pad = N + jnp.arange(EP - E, dtype=jnp.int32) % (NP - N)
    src_p = jnp.concatenate([src, pad]).reshape(NC, NS, CPS, CHUNK)
    dst_p = jnp.concatenate([dst, pad])
    dst_h = dst_p.reshape(NC, NS, EPS)
    dst_c = dst_p.reshape(NC, NS, CPS, CHUNK)
    xp = jnp.pad(x, ((0, NP - N), (0, 0)))

    deg_parts = _make_deg_kernel()(dst_h)
    degt = deg_parts.reshape(NC * NS, NP).T

    g1 = pl.pallas_call(
        _tc_pre_body,
        out_shape=jax.ShapeDtypeStruct((NP, HIDDEN), f32),
    )(xp, W1, degt)

    s1 = _make_scatter_kernel(HIDDEN)(
        g1, src_p, dst_c, jnp.zeros((NP, HIDDEN), f32))

    g2 = pl.pallas_call(
        _tc_mid_body,
        out_shape=jax.ShapeDtypeStruct((NP, HID2), f32),
    )(s1, g1, degt, W2, b1.reshape(1, HIDDEN))

    s2 = _make_scatter_kernel(HID2)(
        g2, src_p, dst_c, jnp.zeros((NP, HID2), f32))

    out = pl.pallas_call(
        _tc_post_body,
        out_shape=jax.ShapeDtypeStruct((NP, 1), f32),
    )(s2, g2, degt, b2.reshape(1, HID2), Wh, bh.reshape(1, 1))

    return out[:N, 0]


# speedup vs baseline: 3.5885x; 1.3907x over previous
import functools

import jax
import jax.numpy as jnp
from jax import lax
from jax.experimental import pallas as pl
from jax.experimental.pallas import tpu as pltpu
from jax.experimental.pallas import tpu_sc as plsc

N = 10000
E = 320000
IN_CH = 128
HIDDEN = 128
HID2 = 64

NC = 2
NS = 16
NP = 10112
RPS = NP // NS
CHUNK = 128
CPS = 80
EPS = CHUNK * CPS
EP = EPS * NC * NS
HSTEP = EPS // 16
HCPS = CPS // 2


def _deg_body(dst_hbm, deg_out, idx_v, hist_v):
    c = lax.axis_index("c")
    s = lax.axis_index("s")
    pltpu.sync_copy(dst_hbm.at[c, s], idx_v)

    def zero_body(i, carry):
        hist_v[pl.ds(i * 16, 16)] = jnp.zeros((16,), jnp.float32)
        return carry

    lax.fori_loop(0, NP // 16, zero_body, 0)

    ones = jnp.ones((16,), jnp.float32)

    def hist_body(i, carry):
        idx16 = idx_v[pl.ds(i * 16, 16)]
        plsc.addupdate_scatter(hist_v, [idx16], ones)
        return carry

    lax.fori_loop(0, HSTEP, hist_body, 0)
    pltpu.sync_copy(hist_v, deg_out.at[c, s])


_SC_PARAMS = pltpu.CompilerParams(needs_layout_passes=False,
                                  use_tc_tiling_on_sc=False)


def _make_deg_kernel():
    return pl.kernel(
        _deg_body,
        out_type=jax.ShapeDtypeStruct((NC, NS, NP), jnp.float32),
        mesh=plsc.VectorSubcoreMesh(core_axis_name="c", subcore_axis_name="s"),
        scratch_types=[
            pltpu.VMEM((EPS,), jnp.int32),
            pltpu.VMEM((NP,), jnp.float32),
        ],
        compiler_params=_SC_PARAMS,
    )


def _scatter_body(g_hbm, src_hbm, dst_hbm, zeros_hbm, s_out,
                  idx_s, idx_d, rows0, rows1, acc, sem0, sem1):
    c = lax.axis_index("c")
    s = lax.axis_index("s")
    pltpu.sync_copy(zeros_hbm.at[pl.ds(s * RPS, RPS)],
                    acc.at[pl.ds(s * RPS, RPS)])
    plsc.subcore_barrier()

    for h in range(CPS // HCPS):
        pltpu.sync_copy(src_hbm.at[c, s, pl.ds(h * HCPS, HCPS)], idx_s)
        pltpu.sync_copy(dst_hbm.at[c, s, pl.ds(h * HCPS, HCPS)], idx_d)
        pltpu.async_copy(g_hbm.at[idx_s.at[0]], rows0, sem0)
        pltpu.async_copy(g_hbm.at[idx_s.at[1]], rows1, sem1)

        def chunk_body(i, carry):
            j0 = 2 * i
            j1 = j0 + 1
            pltpu.make_async_copy(g_hbm.at[idx_s.at[0]], rows0, sem0).wait()
            pltpu.sync_copy(rows0, acc.at[idx_d.at[j0]], add=True)

            @pl.when(j1 + 1 < HCPS)
            def _():
                pltpu.async_copy(g_hbm.at[idx_s.at[j1 + 1]], rows0, sem0)

            pltpu.make_async_copy(g_hbm.at[idx_s.at[0]], rows1, sem1).wait()
            pltpu.sync_copy(rows1, acc.at[idx_d.at[j1]], add=True)

            @pl.when(j1 + 2 < HCPS)
            def _():
                pltpu.async_copy(g_hbm.at[idx_s.at[j1 + 2]], rows1, sem1)

            return carry

        lax.fori_loop(0, HCPS // 2, chunk_body, 0)
    plsc.subcore_barrier()
    pltpu.sync_copy(acc.at[pl.ds(s * RPS, RPS)],
                    s_out.at[c, pl.ds(s * RPS, RPS)])


def _make_scatter_kernel(d):
    return pl.kernel(
        functools.partial(_scatter_body),
        out_type=jax.ShapeDtypeStruct((NC, NP, d), jnp.float32),
        mesh=plsc.VectorSubcoreMesh(core_axis_name="c", subcore_axis_name="s"),
        scratch_types=[
            pltpu.VMEM((HCPS, CHUNK), jnp.int32),
            pltpu.VMEM((HCPS, CHUNK), jnp.int32),
            pltpu.VMEM((CHUNK, d), jnp.float32),
            pltpu.VMEM((CHUNK, d), jnp.float32),
            pltpu.VMEM_SHARED((NP, d), jnp.float32),
            pltpu.SemaphoreType.DMA,
            pltpu.SemaphoreType.DMA,
        ],
        compiler_params=_SC_PARAMS,
    )


def _tc_pre_body(xp_ref, w1_ref, degt_ref, g1_ref):
    deg = jnp.sum(degt_ref[...], axis=1, keepdims=True) + 1.0
    dinv = lax.rsqrt(deg)
    h = lax.dot_general(xp_ref[...], w1_ref[...], (((1,), (1,)), ((), ())),
                        preferred_element_type=jnp.float32)
    g1_ref[...] = h * dinv


def _tc_mid_body(s1_ref, g1_ref, degt_ref, w2_ref, b1_ref, g2_ref):
    deg = jnp.sum(degt_ref[...], axis=1, keepdims=True) + 1.0
    dinv = lax.rsqrt(deg)
    agg = dinv * (s1_ref[0] + s1_ref[1] + g1_ref[...]) + b1_ref[...]
    h1 = jnp.maximum(agg, 0.0)
    h2 = lax.dot_general(h1, w2_ref[...], (((1,), (1,)), ((), ())),
                         preferred_element_type=jnp.float32)
    g2_ref[...] = h2 * dinv


def _tc_post_body(s2_ref, g2_ref, degt_ref, b2_ref, wh_ref, bh_ref, out_ref):
    deg = jnp.sum(degt_ref[...], axis=1, keepdims=True) + 1.0
    dinv = lax.rsqrt(deg)
    agg = dinv * (s2_ref[0] + s2_ref[1] + g2_ref[...]) + b2_ref[...]
    h2 = jnp.maximum(agg, 0.0)
    h2b = h2.astype(jnp.bfloat16).astype(jnp.float32)
    whb = wh_ref[...].astype(jnp.bfloat16).astype(jnp.float32)
    out_ref[...] = jnp.sum(h2b * whb, axis=1, keepdims=True) + bh_ref[...]


def kernel(x, edge_index, W1, b1, W2, b2, Wh, bh):
    f32 = jnp.float32
    src = edge_index[0]
    dst = edge_index[1]
    pad = N + jnp.arange(EP - E, dtype=jnp.int32) % (NP - N)
    src_p = jnp.concatenate([src, pad]).reshape(NC, NS, CPS, CHUNK)
    dst_p = jnp.concatenate([dst, pad])
    dst_h = dst_p.reshape(NC, NS, EPS)
    dst_c = dst_p.reshape(NC, NS, CPS, CHUNK)
    xp = jnp.pad(x, ((0, NP - N), (0, 0)))

    deg_parts = _make_deg_kernel()(dst_h)
    degt = deg_parts.reshape(NC * NS, NP).T

    g1 = pl.pallas_call(
        _tc_pre_body,
        out_shape=jax.ShapeDtypeStruct((NP, HIDDEN), f32),
    )(xp, W1, degt)

    s1 = _make_scatter_kernel(HIDDEN)(
        g1, src_p, dst_c, jnp.zeros((NP, HIDDEN), f32))

    g2 = pl.pallas_call(
        _tc_mid_body,
        out_shape=jax.ShapeDtypeStruct((NP, HID2), f32),
    )(s1, g1, degt, W2, b1.reshape(1, HIDDEN))

    s2 = _make_scatter_kernel(HID2)(
        g2, src_p, dst_c, jnp.zeros((NP, HID2), f32))

    out = pl.pallas_call(
        _tc_post_body,
        out_shape=jax.ShapeDtypeStruct((NP, 1), f32),
    )(s2, g2, degt, b2.reshape(1, HID2), Wh, bh.reshape(1, 1))

    return out[:N, 0]
